# baseline (speedup 1.0000x reference)
import functools

import jax
import jax.numpy as jnp
from jax.experimental import pallas as pl
from jax.experimental.pallas import tpu as pltpu

BB = 32


def _gelu(x):
    half = x * 0.5
    mx = (-x) * 0.707106769
    ax = jnp.abs(mx)
    x2 = mx * mx
    p = x2 * 7.85386146e-05 + -0.000801019371
    p = p * x2 + 0.00518832775
    p = p * x2 + -0.0268538129
    p = p * x2 + 0.112835854
    p = p * x2 + -0.37612626
    p = p * x2 + 1.12837911
    small = 1.0 - mx * p
    z = -x2
    e = jnp.exp(z)
    y = e * (1.0 / ax)
    w = 1.0 / x2
    p2 = w * 0.0232682 + -0.138703942
    p2 = p2 * w + 0.368742466
    p2 = p2 * w + -0.582473278
    p2 = p2 * w + 0.621000469
    p2 = p2 * w + -0.494451523
    p2 = p2 * w + 0.340488
    p2 = p2 * w + -0.274112701
    p2 = p2 * w + 0.563825965
    p3 = w * -10.477664 + 12.9772
    p3 = p3 * w + -7.49551868
    p3 = p3 * w + 2.92101908
    p3 = p3 * w + -1.01526523
    p3 = p3 * w + 0.42184633
    p3 = p3 * w + -0.282076746
    p3 = p3 * w + 0.564189494
    big = y * jnp.where(ax < 2.0, p2, p3)
    big = jnp.where(z < -88.7228394, 0.0, big)
    big = jnp.where(mx < 0.0, 2.0 - big, big)
    return half * jnp.where(ax < 1.0, small, big)


def _gelu_into(dst, x):
    mx = (-x) * 0.707106769
    x2 = mx * mx
    smin = jnp.min(x2)

    @pl.when(smin > 88.7228394)
    def _():
        dst[...] = (x * 0.5) * jnp.where(x > 0.0, 2.0, 0.0)

    @pl.when(jnp.logical_not(smin > 88.7228394))
    def _():
        dst[...] = _gelu(x)


def _block_kernel(T, S, D, DEPTH,
                  xb_ref, mem_any, W1_ref, b1_ref, W2_ref, b2_ref,
                  Wi_ref, bi_ref, Wb1_ref, bb1_ref, Wb2_ref, bb2_ref,
                  Wb3_ref, bb3_ref, wo_ref, gate_ref,
                  out_ref, marg_out_ref,
                  memf, membf, scores_ref, ctx_ref, hid_ref, hbf_ref,
                  marg_ref, gel_ref, sem):
    i = pl.program_id(0)
    cp = pltpu.make_async_copy(mem_any.at[:, pl.ds(i * BB, BB), :], memf, sem)
    cp.start()
    cp.wait()

    def _cv(s, c):
        membf[pl.ds(s * 32, 32)] = memf[pl.ds(s * 32, 32)].astype(
            jnp.bfloat16).astype(jnp.float32)
        return c
    jax.lax.fori_loop(0, S // 32, _cv, 0)

    hid_ref[...] = jnp.zeros((BB, D), jnp.float32)
    marg_ref[...] = jnp.zeros((1, BB), jnp.float32)
    gate = gate_ref[0, 0]

    def _step(t, c):
        h = hid_ref[...]
        xt = xb_ref[pl.ds(t, 1), :, :].reshape(BB, 1)
        h = (h + xt * Wi_ref[...]) + bi_ref[...]
        for l in range(DEPTH):
            u = jnp.dot(h.astype(jnp.bfloat16), W1_ref[l],
                        preferred_element_type=jnp.float32)
            _gelu_into(gel_ref, u + b1_ref[l:l + 1, :])
            v = jnp.dot(gel_ref[...].astype(jnp.bfloat16), W2_ref[l],
                        preferred_element_type=jnp.float32)
            h = (h + v) + b2_ref[l:l + 1, :]
        hbf32 = h.astype(jnp.bfloat16).astype(jnp.float32)

        for s in range(S // 32):
            mslab = membf[s * 32:(s + 1) * 32]
            scores_ref[s * 32:(s + 1) * 32, :] = jnp.sum(
                mslab * hbf32[None, :, :], axis=2)

        sc = scores_ref[...]
        m1 = jnp.max(sc, axis=0, keepdims=True)
        iota = jax.lax.broadcasted_iota(jnp.int32, (S, BB), 0)
        idx = jnp.min(jnp.where(sc == m1, iota, S), axis=0, keepdims=True)
        masked = jnp.where(iota == idx, -jnp.inf, sc)
        m2 = jnp.max(masked, axis=0, keepdims=True)
        marg_ref[...] = marg_ref[...] + (m1 - m2)

        ibs = [idx[0, b] for b in range(BB)]
        for b in range(BB):
            ctx_ref[b:b + 1, :] = memf[pl.ds(ibs[b], 1), b, :]
        ctx = ctx_ref[...]
        u1 = jnp.dot(ctx.astype(jnp.bfloat16), Wb1_ref[...],
                     preferred_element_type=jnp.float32) + bb1_ref[...]
        g1 = _gelu(u1)
        u2 = jnp.dot(g1.astype(jnp.bfloat16), Wb2_ref[...],
                     preferred_element_type=jnp.float32) + bb2_ref[...]
        g2 = _gelu(u2)
        z = jnp.dot(g2.astype(jnp.bfloat16), Wb3_ref[...],
                    preferred_element_type=jnp.float32) + bb3_ref[...]
        h = h + z * gate
        hid_ref[...] = h
        hb2 = h.astype(jnp.bfloat16)
        hbf_ref[...] = hb2.astype(jnp.float32)
        for b in range(BB):
            memf[pl.ds(ibs[b], 1), b, :] = hid_ref[b:b + 1, :]
            membf[pl.ds(ibs[b], 1), b, :] = hbf_ref[b:b + 1, :]
        o = jnp.sum(hbf_ref[...] * wo_ref[...], axis=1, keepdims=True)
        out_ref[pl.ds(t, 1), :, :] = o.reshape(1, BB, 1)
        return c
    jax.lax.fori_loop(0, T, _step, 0)
    marg_out_ref[...] = (marg_ref[...] * jnp.float32(1.0 / T)).reshape(
        1, 1, BB)


def kernel(x, mem0, Wi, bi, W1, b1, W2, b2, Wb1, bb1, Wb2, bb2, Wb3, bb3,
           zoom_gate, Wo, bo):
    B, T = x.shape
    D = Wi.shape[0]
    S = mem0.shape[1]
    BN = Wb1.shape[1]
    DEPTH = W1.shape[0]
    NB = B // BB

    xb = x.T[:, :, None]
    memT = jnp.transpose(mem0, (1, 0, 2))
    W1b = W1.astype(jnp.bfloat16)
    W2b = W2.astype(jnp.bfloat16)
    Wb1b = Wb1.astype(jnp.bfloat16)
    Wb2b = Wb2.astype(jnp.bfloat16)
    Wb3b = Wb3.astype(jnp.bfloat16)
    wo_row = Wo.astype(jnp.bfloat16).astype(jnp.float32).reshape(1, D)
    gate = jax.nn.sigmoid(zoom_gate).reshape(1, 1)
    Wi_row = Wi.reshape(1, D)
    bi_row = bi.reshape(1, D)
    bb1r = bb1.reshape(1, BN)
    bb2r = bb2.reshape(1, BN)
    bb3r = bb3.reshape(1, D)

    body = functools.partial(_block_kernel, T, S, D, DEPTH)
    out_tb, marg = pl.pallas_call(
        body,
        grid=(NB,),
        in_specs=[
            pl.BlockSpec((T, BB, 1), lambda i: (0, i, 0)),
            pl.BlockSpec(memory_space=pl.ANY),
            pl.BlockSpec((DEPTH, D, D), lambda i: (0, 0, 0)),
            pl.BlockSpec((DEPTH, D), lambda i: (0, 0)),
            pl.BlockSpec((DEPTH, D, D), lambda i: (0, 0, 0)),
            pl.BlockSpec((DEPTH, D), lambda i: (0, 0)),
            pl.BlockSpec((1, D), lambda i: (0, 0)),
            pl.BlockSpec((1, D), lambda i: (0, 0)),
            pl.BlockSpec((D, BN), lambda i: (0, 0)),
            pl.BlockSpec((1, BN), lambda i: (0, 0)),
            pl.BlockSpec((BN, BN), lambda i: (0, 0)),
            pl.BlockSpec((1, BN), lambda i: (0, 0)),
            pl.BlockSpec((BN, D), lambda i: (0, 0)),
            pl.BlockSpec((1, D), lambda i: (0, 0)),
            pl.BlockSpec((1, D), lambda i: (0, 0)),
            pl.BlockSpec(memory_space=pltpu.SMEM),
        ],
        out_specs=[
            pl.BlockSpec((T, BB, 1), lambda i: (0, i, 0)),
            pl.BlockSpec((1, 1, BB), lambda i: (i, 0, 0)),
        ],
        out_shape=[
            jax.ShapeDtypeStruct((T, B, 1), jnp.float32),
            jax.ShapeDtypeStruct((NB, 1, BB), jnp.float32),
        ],
        scratch_shapes=[
            pltpu.VMEM((S, BB, D), jnp.float32),
            pltpu.VMEM((S, BB, D), jnp.float32),
            pltpu.VMEM((S, BB), jnp.float32),
            pltpu.VMEM((BB, D), jnp.float32),
            pltpu.VMEM((BB, D), jnp.float32),
            pltpu.VMEM((BB, D), jnp.float32),
            pltpu.VMEM((1, BB), jnp.float32),
            pltpu.VMEM((BB, D), jnp.float32),
            pltpu.SemaphoreType.DMA,
        ],
        compiler_params=pltpu.CompilerParams(
            dimension_semantics=("arbitrary",),
            vmem_limit_bytes=56 * 1024 * 1024,
        ),
    )(xb, memT, W1b, b1, W2b, b2, Wi_row, bi_row, Wb1b, bb1r, Wb2b, bb2r,
      Wb3b, bb3r, wo_row, gate)

    outs = (out_tb[:, :, 0] + bo[0]).T
    return outs, marg.reshape(B)


# speedup vs baseline: 1.1492x; 1.1492x over previous
import functools

import jax
import jax.numpy as jnp
from jax.experimental import pallas as pl
from jax.experimental.pallas import tpu as pltpu

BB = 32


def _gelu(x):
    half = x * 0.5
    mx = (-x) * 0.707106769
    ax = jnp.abs(mx)
    x2 = mx * mx
    p = x2 * 7.85386146e-05 + -0.000801019371
    p = p * x2 + 0.00518832775
    p = p * x2 + -0.0268538129
    p = p * x2 + 0.112835854
    p = p * x2 + -0.37612626
    p = p * x2 + 1.12837911
    small = 1.0 - mx * p
    z = -x2
    e = jnp.exp(z)
    y = e * (1.0 / ax)
    w = 1.0 / x2
    p2 = w * 0.0232682 + -0.138703942
    p2 = p2 * w + 0.368742466
    p2 = p2 * w + -0.582473278
    p2 = p2 * w + 0.621000469
    p2 = p2 * w + -0.494451523
    p2 = p2 * w + 0.340488
    p2 = p2 * w + -0.274112701
    p2 = p2 * w + 0.563825965
    p3 = w * -10.477664 + 12.9772
    p3 = p3 * w + -7.49551868
    p3 = p3 * w + 2.92101908
    p3 = p3 * w + -1.01526523
    p3 = p3 * w + 0.42184633
    p3 = p3 * w + -0.282076746
    p3 = p3 * w + 0.564189494
    big = y * jnp.where(ax < 2.0, p2, p3)
    big = jnp.where(z < -88.7228394, 0.0, big)
    big = jnp.where(mx < 0.0, 2.0 - big, big)
    return half * jnp.where(ax < 1.0, small, big)


def _block_kernel(T, S, D, DEPTH,
                  xb_ref, mem_any, W1_ref, b1_ref, W2_ref, b2_ref,
                  Wi_ref, bi_ref, Wb1_ref, bb1_ref, Wb2_ref, bb2_ref,
                  Wb3_ref, bb3_ref, wo_ref, gate_ref,
                  out_ref, marg_out_ref,
                  memf, membf, scores_ref, ctx_ref, hid_ref, hbf_ref,
                  marg_ref, sem):
    i = pl.program_id(0)
    cp = pltpu.make_async_copy(mem_any.at[:, pl.ds(i * BB, BB), :], memf, sem)
    cp.start()
    cp.wait()

    def _cv(s, c):
        membf[pl.ds(s * 32, 32)] = memf[pl.ds(s * 32, 32)].astype(
            jnp.bfloat16).astype(jnp.float32)
        return c
    jax.lax.fori_loop(0, S // 32, _cv, 0)

    hid_ref[...] = jnp.zeros((BB, D), jnp.float32)
    marg_ref[...] = jnp.zeros((1, BB), jnp.float32)
    gate = gate_ref[0, 0]

    def _step(t, c):
        h = hid_ref[...]
        xt = xb_ref[pl.ds(t, 1), :, :].reshape(BB, 1)
        h = (h + xt * Wi_ref[...]) + bi_ref[...]
        for l in range(DEPTH):
            u = jnp.dot(h.astype(jnp.bfloat16), W1_ref[l],
                        preferred_element_type=jnp.float32)
            g = _gelu(u + b1_ref[l:l + 1, :])
            v = jnp.dot(g.astype(jnp.bfloat16), W2_ref[l],
                        preferred_element_type=jnp.float32)
            h = (h + v) + b2_ref[l:l + 1, :]
        hbf32 = h.astype(jnp.bfloat16).astype(jnp.float32)

        for s in range(S // 32):
            mslab = membf[s * 32:(s + 1) * 32]
            scores_ref[s * 32:(s + 1) * 32, :] = jnp.sum(
                mslab * hbf32[None, :, :], axis=2)

        sc = scores_ref[...]
        m1 = jnp.max(sc, axis=0, keepdims=True)
        iota = jax.lax.broadcasted_iota(jnp.int32, (S, BB), 0)
        idx = jnp.min(jnp.where(sc == m1, iota, S), axis=0, keepdims=True)
        masked = jnp.where(iota == idx, -jnp.inf, sc)
        m2 = jnp.max(masked, axis=0, keepdims=True)
        marg_ref[...] = marg_ref[...] + (m1 - m2)

        ibs = [idx[0, b] for b in range(BB)]
        for b in range(BB):
            ctx_ref[b:b + 1, :] = memf[pl.ds(ibs[b], 1), b, :]
        ctx = ctx_ref[...]
        u1 = jnp.dot(ctx.astype(jnp.bfloat16), Wb1_ref[...],
                     preferred_element_type=jnp.float32) + bb1_ref[...]
        g1 = _gelu(u1)
        u2 = jnp.dot(g1.astype(jnp.bfloat16), Wb2_ref[...],
                     preferred_element_type=jnp.float32) + bb2_ref[...]
        g2 = _gelu(u2)
        z = jnp.dot(g2.astype(jnp.bfloat16), Wb3_ref[...],
                    preferred_element_type=jnp.float32) + bb3_ref[...]
        h = h + z * gate
        hid_ref[...] = h
        hb2 = h.astype(jnp.bfloat16)
        hbf_ref[...] = hb2.astype(jnp.float32)
        for b in range(BB):
            memf[pl.ds(ibs[b], 1), b, :] = hid_ref[b:b + 1, :]
            membf[pl.ds(ibs[b], 1), b, :] = hbf_ref[b:b + 1, :]
        o = jnp.sum(hbf_ref[...] * wo_ref[...], axis=1, keepdims=True)
        out_ref[pl.ds(t, 1), :, :] = o.reshape(1, BB, 1)
        return c
    jax.lax.fori_loop(0, T, _step, 0)
    marg_out_ref[...] = (marg_ref[...] * jnp.float32(1.0 / T)).reshape(
        1, 1, BB)


def kernel(x, mem0, Wi, bi, W1, b1, W2, b2, Wb1, bb1, Wb2, bb2, Wb3, bb3,
           zoom_gate, Wo, bo):
    B, T = x.shape
    D = Wi.shape[0]
    S = mem0.shape[1]
    BN = Wb1.shape[1]
    DEPTH = W1.shape[0]
    NB = B // BB

    xb = x.T[:, :, None]
    memT = jnp.transpose(mem0, (1, 0, 2))
    W1b = W1.astype(jnp.bfloat16)
    W2b = W2.astype(jnp.bfloat16)
    Wb1b = Wb1.astype(jnp.bfloat16)
    Wb2b = Wb2.astype(jnp.bfloat16)
    Wb3b = Wb3.astype(jnp.bfloat16)
    wo_row = Wo.astype(jnp.bfloat16).astype(jnp.float32).reshape(1, D)
    gate = jax.nn.sigmoid(zoom_gate).reshape(1, 1)
    Wi_row = Wi.reshape(1, D)
    bi_row = bi.reshape(1, D)
    bb1r = bb1.reshape(1, BN)
    bb2r = bb2.reshape(1, BN)
    bb3r = bb3.reshape(1, D)

    body = functools.partial(_block_kernel, T, S, D, DEPTH)
    out_tb, marg = pl.pallas_call(
        body,
        grid=(NB,),
        in_specs=[
            pl.BlockSpec((T, BB, 1), lambda i: (0, i, 0)),
            pl.BlockSpec(memory_space=pl.ANY),
            pl.BlockSpec((DEPTH, D, D), lambda i: (0, 0, 0)),
            pl.BlockSpec((DEPTH, D), lambda i: (0, 0)),
            pl.BlockSpec((DEPTH, D, D), lambda i: (0, 0, 0)),
            pl.BlockSpec((DEPTH, D), lambda i: (0, 0)),
            pl.BlockSpec((1, D), lambda i: (0, 0)),
            pl.BlockSpec((1, D), lambda i: (0, 0)),
            pl.BlockSpec((D, BN), lambda i: (0, 0)),
            pl.BlockSpec((1, BN), lambda i: (0, 0)),
            pl.BlockSpec((BN, BN), lambda i: (0, 0)),
            pl.BlockSpec((1, BN), lambda i: (0, 0)),
            pl.BlockSpec((BN, D), lambda i: (0, 0)),
            pl.BlockSpec((1, D), lambda i: (0, 0)),
            pl.BlockSpec((1, D), lambda i: (0, 0)),
            pl.BlockSpec(memory_space=pltpu.SMEM),
        ],
        out_specs=[
            pl.BlockSpec((T, BB, 1), lambda i: (0, i, 0)),
            pl.BlockSpec((1, 1, BB), lambda i: (i, 0, 0)),
        ],
        out_shape=[
            jax.ShapeDtypeStruct((T, B, 1), jnp.float32),
            jax.ShapeDtypeStruct((NB, 1, BB), jnp.float32),
        ],
        scratch_shapes=[
            pltpu.VMEM((S, BB, D), jnp.float32),
            pltpu.VMEM((S, BB, D), jnp.float32),
            pltpu.VMEM((S, BB), jnp.float32),
            pltpu.VMEM((BB, D), jnp.float32),
            pltpu.VMEM((BB, D), jnp.float32),
            pltpu.VMEM((BB, D), jnp.float32),
            pltpu.VMEM((1, BB), jnp.float32),
            pltpu.SemaphoreType.DMA,
        ],
        compiler_params=pltpu.CompilerParams(
            dimension_semantics=("arbitrary",),
            vmem_limit_bytes=56 * 1024 * 1024,
        ),
    )(xb, memT, W1b, b1, W2b, b2, Wi_row, bi_row, Wb1b, bb1r, Wb2b, bb2r,
      Wb3b, bb3r, wo_row, gate)

    outs = (out_tb[:, :, 0] + bo[0]).T
    return outs, marg.reshape(B)
